# Initial kernel scaffold; baseline (speedup 1.0000x reference)
import jax
import jax.numpy as jnp
from jax import lax
from jax.experimental import pallas as pl
from jax.experimental.pallas import tpu as pltpu
from jax.experimental.pallas import tpu_sc as plsc

N = 10000
D = 128
H = 64
G = 64
NC = 10

NCORES = 2
NSUB = 16
NW = NCORES * NSUB
LANES = 16

CHUNK = 256
NCH = 41
EPT = NCH * CHUNK
EPAD = NW * EPT
E_REAL = 320000 + N
RPT = N // NSUB


def _tc_pre_body(x_ref, w_ref, as_ref, ad_ref, h_ref, att_ref):
    h = jnp.dot(x_ref[...], w_ref[...], preferred_element_type=jnp.float32)
    h_ref[...] = h
    a_s = jnp.sum(h * as_ref[...], axis=-1)
    a_d = jnp.sum(h * ad_ref[...], axis=-1)
    att_ref[...] = jnp.stack([a_s, a_d])


def _tc_pre(x, w, att_s, att_d):
    n = x.shape[0]
    return pl.pallas_call(
        _tc_pre_body,
        out_shape=[
            jax.ShapeDtypeStruct((n, w.shape[1]), jnp.float32),
            jax.ShapeDtypeStruct((2, n), jnp.float32),
        ],
    )(x, w, att_s.reshape(1, -1), att_d.reshape(1, -1))


def _tc_mid_body(p_ref, d_ref, b_ref, w_ref, as_ref, ad_ref, h_ref, att_ref):
    num = p_ref[0:N, :] + p_ref[N:2 * N, :]
    den = jnp.sum(d_ref[...], axis=0) + 1e-16
    h1 = jnp.maximum(num / den[:, None] + b_ref[...], 0.0)
    h = jnp.dot(h1, w_ref[...], preferred_element_type=jnp.float32)
    h_ref[...] = h
    a_s = jnp.sum(h * as_ref[...], axis=-1)
    a_d = jnp.sum(h * ad_ref[...], axis=-1)
    att_ref[...] = jnp.stack([a_s, a_d])


def _tc_mid(p, dpart, b1, w2, att_s2, att_d2):
    return pl.pallas_call(
        _tc_mid_body,
        out_shape=[
            jax.ShapeDtypeStruct((N, H), jnp.float32),
            jax.ShapeDtypeStruct((2, N), jnp.float32),
        ],
    )(p, dpart, b1.reshape(1, -1), w2,
      att_s2.reshape(1, -1), att_d2.reshape(1, -1))


def _tc_head_body(p_ref, d_ref, b_ref, batch_ref, wl_ref, bl_ref, wc_ref,
                  bc_ref, out_ref):
    num = p_ref[0:N, :] + p_ref[N:2 * N, :]
    den = jnp.sum(d_ref[...], axis=0) + 1e-16
    h = num / den[:, None] + b_ref[...]
    gids = lax.broadcasted_iota(jnp.int32, (G, N), 0)
    onehot = (gids == batch_ref[...]).astype(jnp.float32)
    sums = jnp.dot(onehot, h, preferred_element_type=jnp.float32)
    counts = jnp.sum(onehot, axis=1)
    pooled = sums / jnp.maximum(counts, 1.0)[:, None]
    z = jnp.maximum(jnp.dot(pooled, wl_ref[...],
                            preferred_element_type=jnp.float32)
                    + bl_ref[...], 0.0)
    logits = jnp.dot(z, wc_ref[...],
                     preferred_element_type=jnp.float32) + bc_ref[...]
    shifted = logits - jnp.max(logits, axis=1, keepdims=True)
    lse = jnp.log(jnp.sum(jnp.exp(shifted), axis=1, keepdims=True))
    out_ref[...] = shifted - lse


def _tc_head(p, dpart, b2, batch, w_lin, b_lin, w_cls, b_cls):
    return pl.pallas_call(
        _tc_head_body,
        out_shape=jax.ShapeDtypeStruct((G, NC), jnp.float32),
    )(p, dpart, b2.reshape(1, -1), batch.reshape(1, -1),
      w_lin, b_lin.reshape(1, -1), w_cls, b_cls.reshape(1, -1))


def _sc_edge_body(src_hbm, dst_hbm, att_hbm, h_hbm,
                  p_hbm, dpart_hbm,
                  srcv, dstv, asv, adv, exv, denv, hbuf, mbuf, shared_out):
    cid = lax.axis_index("c")
    sid = lax.axis_index("s")
    wid = cid * NSUB + sid

    pltpu.sync_copy(src_hbm.at[wid], srcv)
    pltpu.sync_copy(dst_hbm.at[wid], dstv)
    pltpu.sync_copy(att_hbm.at[0], asv)
    pltpu.sync_copy(att_hbm.at[1], adv)

    zeros16 = jnp.zeros((LANES,), jnp.float32)

    @pl.loop(0, N, step=LANES)
    def _(i):
        denv[pl.ds(i, LANES)] = zeros16

    @pl.loop(0, CHUNK)
    def _(r):
        row = hbuf.at[r]
        for cg in range(H // LANES):
            row[pl.ds(cg * LANES, LANES)] = zeros16

    for kk in range(5):
        pltpu.sync_copy(
            hbuf.at[pl.ds(0, 125)],
            shared_out.at[pl.ds(sid * RPT + kk * 125, 125)])

    neg = jnp.full((LANES,), -3.0e38, jnp.float32)
    mbuf[...] = neg

    @pl.loop(0, N, step=LANES)
    def _(i):
        mbuf[...] = jnp.maximum(mbuf[...], asv[pl.ds(i, LANES)])

    ms = jnp.max(mbuf[...])
    mbuf[...] = neg

    @pl.loop(0, N, step=LANES)
    def _(i):
        mbuf[...] = jnp.maximum(mbuf[...], adv[pl.ds(i, LANES)])

    md = jnp.max(mbuf[...])
    mlb = ms + md
    c = jnp.maximum(mlb, 0.2 * mlb)
    cvec = lax.broadcast(c, (LANES,))

    iota16 = lax.iota(jnp.int32, LANES)
    base0 = wid * EPT

    @pl.loop(0, NCH)
    def _(j):
        srow = srcv.at[j]
        drow = dstv.at[j]
        erow = exv.at[j]
        cbase = base0 + j * CHUNK

        @pl.loop(0, CHUNK, step=LANES)
        def _(i):
            sidx = srow[pl.ds(i, LANES)]
            didx = drow[pl.ds(i, LANES)]
            s = plsc.load_gather(asv, [sidx])
            d = plsc.load_gather(adv, [didx])
            t = s + d
            t = jnp.maximum(t, 0.2 * t) - cvec
            e = jnp.exp(t)
            gid = lax.broadcast(cbase + i, (LANES,)) + iota16
            e = jnp.where(gid < E_REAL, e, 0.0)
            erow[pl.ds(i, LANES)] = e
            plsc.addupdate_scatter(denv, [didx], e)

    pltpu.sync_copy(denv, dpart_hbm.at[wid])

    plsc.subcore_barrier()

    @pl.loop(0, NCH)
    def _(j):
        pltpu.sync_copy(h_hbm.at[srcv.at[j]], hbuf)

        erow = exv.at[j]

        @pl.loop(0, CHUNK)
        def _(e):
            ev = plsc.load_gather(erow, [lax.broadcast(e, (LANES,))])
            row = hbuf.at[e]
            for cg in range(H // LANES):
                sl = pl.ds(cg * LANES, LANES)
                row[sl] = row[sl] * ev

        pltpu.sync_copy(hbuf, shared_out.at[dstv.at[j]], add=True)

    plsc.subcore_barrier()

    pltpu.sync_copy(shared_out.at[pl.ds(sid * RPT, RPT)],
                    p_hbm.at[pl.ds(cid * N + sid * RPT, RPT)])


def _sc_edge(src3, dst3, att, h):
    mesh = plsc.VectorSubcoreMesh(core_axis_name="c", subcore_axis_name="s")
    kern = pl.kernel(
        _sc_edge_body,
        mesh=mesh,
        out_type=[
            jax.ShapeDtypeStruct((NCORES * N, H), jnp.float32),
            jax.ShapeDtypeStruct((NW, N), jnp.float32),
        ],
        scratch_types=[
            pltpu.VMEM((NCH, CHUNK), jnp.int32),
            pltpu.VMEM((NCH, CHUNK), jnp.int32),
            pltpu.VMEM((N,), jnp.float32),
            pltpu.VMEM((N,), jnp.float32),
            pltpu.VMEM((NCH, CHUNK), jnp.float32),
            pltpu.VMEM((N,), jnp.float32),
            pltpu.VMEM((CHUNK, H), jnp.float32),
            pltpu.VMEM((LANES,), jnp.float32),
            pltpu.VMEM_SHARED((N, H), jnp.float32),
        ],
    )
    return kern(src3, dst3, att, h)


def kernel(x, edge_index, batch, W1, att_src1, att_dst1, b1,
           W2, att_src2, att_dst2, b2, W_lin, b_lin, W_cls, b_cls):
    n = x.shape[0]
    loop = jnp.arange(n, dtype=edge_index.dtype)
    src = jnp.concatenate([edge_index[0], loop])
    dst = jnp.concatenate([edge_index[1], loop])
    pad = EPAD - E_REAL
    src3 = jnp.pad(src, (0, pad)).reshape(NW, NCH, CHUNK)
    dst3 = jnp.pad(dst, (0, pad)).reshape(NW, NCH, CHUNK)

    h1, att1 = _tc_pre(x, W1, att_src1, att_dst1)
    p1, d1 = _sc_edge(src3, dst3, att1, h1)
    h2, att2 = _tc_mid(p1, d1, b1, W2, att_src2, att_dst2)
    p2, d2 = _sc_edge(src3, dst3, att2, h2)
    return _tc_head(p2, d2, b2, batch, W_lin, b_lin, W_cls, b_cls)


# speedup vs baseline: 31.2394x; 31.2394x over previous
import dataclasses

import jax
import jax.numpy as jnp
from jax import lax
from jax.experimental import pallas as pl
from jax.experimental.pallas import tpu as pltpu
from jax.experimental.pallas import tpu_sc as plsc

N = 10000
D = 128
H = 64
G = 64
NC = 10

NCORES = 2
NSUB = 16
NW = NCORES * NSUB
LANES = 16

CHUNK = 256
NCH = 41
EPT = NCH * CHUNK
EPAD = NW * EPT
E_REAL = 320000 + N
NPAD = 10240
RPT = NPAD // NSUB
HP = 64


def _tc_pre_body(x_ref, w_ref, as_ref, ad_ref, h_ref, att_ref):
    h = jnp.dot(x_ref[...], w_ref[...], preferred_element_type=jnp.float32)
    h_ref[0:N, 0:H] = h
    if HP > H:
        h_ref[0:N, H:HP] = jnp.zeros((N, HP - H), jnp.float32)
    h_ref[N:NPAD, :] = jnp.zeros((NPAD - N, HP), jnp.float32)
    a_s = jnp.sum(h * as_ref[...], axis=-1)
    a_d = jnp.sum(h * ad_ref[...], axis=-1)
    att_ref[...] = jnp.stack([a_s, a_d])


def _tc_pre(x, w, att_s, att_d):
    n = x.shape[0]
    return pl.pallas_call(
        _tc_pre_body,
        out_shape=[
            jax.ShapeDtypeStruct((NPAD, HP), jnp.float32),
            jax.ShapeDtypeStruct((2, n), jnp.float32),
        ],
    )(x, w, att_s.reshape(1, -1), att_d.reshape(1, -1))


def _tc_mid_body(p_ref, d_ref, b_ref, w_ref, as_ref, ad_ref, h_ref, att_ref):
    num = p_ref[0:N, 0:H] + p_ref[NPAD:NPAD + N, 0:H]
    den = jnp.sum(d_ref[...], axis=0) + 1e-16
    h1 = jnp.maximum(num / den[:, None] + b_ref[...], 0.0)
    h = jnp.dot(h1, w_ref[...], preferred_element_type=jnp.float32)
    h_ref[0:N, 0:H] = h
    if HP > H:
        h_ref[0:N, H:HP] = jnp.zeros((N, HP - H), jnp.float32)
    h_ref[N:NPAD, :] = jnp.zeros((NPAD - N, HP), jnp.float32)
    a_s = jnp.sum(h * as_ref[...], axis=-1)
    a_d = jnp.sum(h * ad_ref[...], axis=-1)
    att_ref[...] = jnp.stack([a_s, a_d])


def _tc_mid(p, dpart, b1, w2, att_s2, att_d2):
    return pl.pallas_call(
        _tc_mid_body,
        out_shape=[
            jax.ShapeDtypeStruct((NPAD, HP), jnp.float32),
            jax.ShapeDtypeStruct((2, N), jnp.float32),
        ],
    )(p, dpart, b1.reshape(1, -1), w2,
      att_s2.reshape(1, -1), att_d2.reshape(1, -1))


def _tc_head_body(p_ref, d_ref, b_ref, batch_ref, wl_ref, bl_ref, wc_ref,
                  bc_ref, out_ref):
    num = p_ref[0:N, 0:H] + p_ref[NPAD:NPAD + N, 0:H]
    den = jnp.sum(d_ref[...], axis=0) + 1e-16
    h = num / den[:, None] + b_ref[...]
    gids = lax.broadcasted_iota(jnp.int32, (G, N), 0)
    onehot = (gids == batch_ref[...]).astype(jnp.float32)
    sums = jnp.dot(onehot, h, preferred_element_type=jnp.float32)
    counts = jnp.sum(onehot, axis=1)
    pooled = sums / jnp.maximum(counts, 1.0)[:, None]
    z = jnp.maximum(jnp.dot(pooled, wl_ref[...],
                            preferred_element_type=jnp.float32)
                    + bl_ref[...], 0.0)
    logits = jnp.dot(z, wc_ref[...],
                     preferred_element_type=jnp.float32) + bc_ref[...]
    shifted = logits - jnp.max(logits, axis=1, keepdims=True)
    lse = jnp.log(jnp.sum(jnp.exp(shifted), axis=1, keepdims=True))
    out_ref[...] = shifted - lse


def _tc_head(p, dpart, b2, batch, w_lin, b_lin, w_cls, b_cls):
    return pl.pallas_call(
        _tc_head_body,
        out_shape=jax.ShapeDtypeStruct((G, NC), jnp.float32),
    )(p, dpart, b2.reshape(1, -1), batch.reshape(1, -1),
      w_lin, b_lin.reshape(1, -1), w_cls, b_cls.reshape(1, -1))


def _sc_edge_body(src_hbm, dst_hbm, att_hbm, h_hbm,
                  p_hbm, dpart_hbm,
                  srcv, dstv, asv, adv, exv, denv, hbuf, mbuf, shared_out):
    cid = lax.axis_index("c")
    sid = lax.axis_index("s")
    wid = cid * NSUB + sid

    pltpu.sync_copy(src_hbm.at[wid], srcv)
    pltpu.sync_copy(dst_hbm.at[wid], dstv)
    pltpu.sync_copy(att_hbm.at[pl.ds(0, N)], asv)
    pltpu.sync_copy(att_hbm.at[pl.ds(N, N)], adv)

    zeros16 = jnp.zeros((LANES,), jnp.float32)
    zrow = jnp.zeros((1, LANES), jnp.float32)

    @pl.loop(0, N, step=LANES)
    def _(i):
        denv[pl.ds(i, LANES)] = zeros16

    @pl.loop(0, CHUNK)
    def _(r):
        for cg in range(HP // LANES):
            hbuf[r, pl.ds(cg * LANES, LANES)] = zeros16

    for kk in range(5):
        pltpu.sync_copy(
            hbuf.at[pl.ds(0, 128)],
            shared_out.at[pl.ds(sid * RPT + kk * 128, 128)])

    neg = jnp.full((LANES,), -3.0e38, jnp.float32)
    mbuf[...] = neg

    @pl.loop(0, N, step=LANES)
    def _(i):
        mbuf[...] = jnp.maximum(mbuf[...], asv[pl.ds(i, LANES)])

    ms = jnp.max(mbuf[...])
    mbuf[...] = neg

    @pl.loop(0, N, step=LANES)
    def _(i):
        mbuf[...] = jnp.maximum(mbuf[...], adv[pl.ds(i, LANES)])

    md = jnp.max(mbuf[...])
    mlb = ms + md
    c = jnp.maximum(mlb, 0.2 * mlb)
    cvec = lax.broadcast(c, (LANES,))

    iota16 = lax.iota(jnp.int32, LANES)
    base0 = wid * EPT

    @pl.loop(0, NCH)
    def _(j):
        cbase = base0 + j * CHUNK

        @pl.loop(0, CHUNK, step=LANES)
        def _(i):
            sidx = srcv[j, pl.ds(i, LANES)]
            didx = dstv[j, pl.ds(i, LANES)]
            s = plsc.load_gather(asv, [sidx])
            d = plsc.load_gather(adv, [didx])
            t = s + d
            t = jnp.maximum(t, 0.2 * t) - cvec
            e = jnp.exp(t)
            gid = lax.broadcast(cbase + i, (LANES,)) + iota16
            e = jnp.where(gid < E_REAL, e, 0.0)
            exv[j, pl.ds(i, LANES)] = e
            plsc.addupdate_scatter(denv, [didx], e)

    pltpu.sync_copy(denv, dpart_hbm.at[pl.ds(wid * N, N)])

    plsc.subcore_barrier()

    @pl.loop(0, NCH)
    def _(j):
        pltpu.sync_copy(h_hbm.at[srcv.at[j]], hbuf)
        jvec = lax.broadcast(j, (LANES,))

        @pl.loop(0, CHUNK)
        def _(e):
            ev = plsc.load_gather(exv, [jvec, lax.broadcast(e, (LANES,))])
            for cg in range(H // LANES):
                sl = pl.ds(cg * LANES, LANES)
                hbuf[e, sl] = hbuf[e, sl] * ev

        pltpu.sync_copy(hbuf, shared_out.at[dstv.at[j]], add=True)

    plsc.subcore_barrier()

    pltpu.sync_copy(shared_out.at[pl.ds(sid * RPT, RPT)],
                    p_hbm.at[pl.ds(cid * NPAD + sid * RPT, RPT)])


def _sc_edge(src3, dst3, att, h):
    mesh = plsc.VectorSubcoreMesh(core_axis_name="c", subcore_axis_name="s")
    cp = pltpu.CompilerParams()
    if "needs_layout_passes" in pltpu.CompilerParams.__dataclass_fields__:
        cp = dataclasses.replace(cp, needs_layout_passes=False)
    if "use_tc_tiling_on_sc" in pltpu.CompilerParams.__dataclass_fields__:
        cp = dataclasses.replace(cp, use_tc_tiling_on_sc=False)
    kern = pl.kernel(
        _sc_edge_body,
        mesh=mesh,
        out_type=[
            jax.ShapeDtypeStruct((NCORES * NPAD, HP), jnp.float32),
            jax.ShapeDtypeStruct((NW * N,), jnp.float32),
        ],
        scratch_types=[
            pltpu.VMEM((NCH, CHUNK), jnp.int32),
            pltpu.VMEM((NCH, CHUNK), jnp.int32),
            pltpu.VMEM((N,), jnp.float32),
            pltpu.VMEM((N,), jnp.float32),
            pltpu.VMEM((NCH, CHUNK), jnp.float32),
            pltpu.VMEM((N,), jnp.float32),
            pltpu.VMEM((CHUNK, HP), jnp.float32),
            pltpu.VMEM((LANES,), jnp.float32),
            pltpu.VMEM_SHARED((NPAD, HP), jnp.float32),
        ],
        compiler_params=cp,
    )
    p, dpart = kern(src3, dst3, att.reshape(2 * N), h)
    return p, dpart.reshape(NW, N)


def kernel(x, edge_index, batch, W1, att_src1, att_dst1, b1,
           W2, att_src2, att_dst2, b2, W_lin, b_lin, W_cls, b_cls):
    n = x.shape[0]
    loop = jnp.arange(n, dtype=edge_index.dtype)
    src = jnp.concatenate([edge_index[0], loop])
    dst = jnp.concatenate([edge_index[1], loop])
    pad = EPAD - E_REAL
    src3 = jnp.pad(src, (0, pad)).reshape(NW, NCH, CHUNK)
    dst3 = jnp.pad(dst, (0, pad)).reshape(NW, NCH, CHUNK)

    h1, att1 = _tc_pre(x, W1, att_src1, att_dst1)
    p1, d1 = _sc_edge(src3, dst3, att1, h1)
    h2, att2 = _tc_mid(p1, d1, b1, W2, att_src2, att_dst2)
    p2, d2 = _sc_edge(src3, dst3, att2, h2)
    return _tc_head(p2, d2, b2, batch, W_lin, b_lin, W_cls, b_cls)
